# baseline (speedup 1.0000x reference)
import jax
import jax.numpy as jnp
from jax.experimental import pallas as pl
from jax.experimental.pallas import tpu as pltpu
from jax.experimental.pallas import tpu_sc as plsc

_VOCAB = 1000000
_D = 64
_NS = 2048
_BR = 1024

_VPAD = 1024 * 1024


def _gumbel_padded():
    u = jax.random.uniform(
        jax.random.key(42), (_VOCAB,), minval=1e-10, maxval=1.0
    )
    g = -jnp.log(-jnp.log(u))
    return jnp.full((_VPAD,), -1e30, jnp.float32).at[:_VOCAB].set(g)


_ROWS = 1024
_COLS = 1024
_NCHUNK = 32


def _select_body(probs_ref, gumbel_ref, keys_ref, offs_ref, tt_ref, ti_ref):
    z = jnp.log(probs_ref[...] + 1e-10) + gumbel_ref[...]
    b = jax.lax.bitcast_convert_type(z, jnp.uint32)
    key = jnp.where((b >> 31) == 1, ~b, b | jnp.uint32(0x80000000))
    keys_ref[...] = jax.lax.bitcast_convert_type(
        key ^ jnp.uint32(0x80000000), jnp.int32
    )

    row = jax.lax.broadcasted_iota(jnp.int32, (_ROWS, _COLS), 0)
    col = jax.lax.broadcasted_iota(jnp.int32, (_ROWS, _COLS), 1)
    idx = row * _COLS + col

    def bit_body(i, t):
        cand = t | jax.lax.shift_left(
            jnp.uint32(1), (31 - i).astype(jnp.uint32)
        )
        cnt = jnp.sum((key >= cand).astype(jnp.int32))
        return jnp.where(cnt >= _NS, cand, t)

    tval = jax.lax.fori_loop(0, 32, bit_body, jnp.uint32(0))

    need = _NS - jnp.sum((key > tval).astype(jnp.int32))

    def tie_body(j, iv):
        cand = iv | jax.lax.shift_left(jnp.int32(1), 19 - j)
        cnt = jnp.sum(((key == tval) & (idx <= cand)).astype(jnp.int32))
        return jnp.where(cnt <= need, cand, iv)

    ival = jax.lax.fori_loop(0, 20, tie_body, jnp.int32(0))

    sel = (key > tval) | ((key == tval) & (idx <= ival))
    rowsum = jnp.sum(sel.astype(jnp.float32), axis=1, keepdims=True)
    amat = (
        (jax.lax.broadcasted_iota(jnp.int32, (_NCHUNK, _ROWS), 1)
         // (_ROWS // _NCHUNK))
        == jax.lax.broadcasted_iota(jnp.int32, (_NCHUNK, _ROWS), 0)
    ).astype(jnp.float32)
    counts = jnp.dot(amat, rowsum, preferred_element_type=jnp.float32)
    ltri = (
        jax.lax.broadcasted_iota(jnp.int32, (_NCHUNK, _NCHUNK), 0)
        > jax.lax.broadcasted_iota(jnp.int32, (_NCHUNK, _NCHUNK), 1)
    ).astype(jnp.float32)
    offs = jnp.dot(ltri, counts, preferred_element_type=jnp.float32)
    offs_ref[...] = offs.astype(jnp.int32)
    t_signed = jax.lax.bitcast_convert_type(
        tval ^ jnp.uint32(0x80000000), jnp.int32
    )
    tt_ref[...] = jnp.full((1, 1), t_signed, dtype=jnp.int32)
    ti_ref[...] = jnp.full((1, 1), ival, dtype=jnp.int32)


def _select(probs2d, gumbel2d):
    return pl.pallas_call(
        _select_body,
        grid=(1,),
        in_specs=[
            pl.BlockSpec((_ROWS, _COLS), lambda i: (0, 0)),
            pl.BlockSpec((_ROWS, _COLS), lambda i: (0, 0)),
        ],
        out_specs=[
            pl.BlockSpec((_ROWS, _COLS), lambda i: (0, 0)),
            pl.BlockSpec((_NCHUNK, 1), lambda i: (0, 0)),
            pl.BlockSpec((1, 1), lambda i: (0, 0)),
            pl.BlockSpec((1, 1), lambda i: (0, 0)),
        ],
        out_shape=[
            jax.ShapeDtypeStruct((_ROWS, _COLS), jnp.int32),
            jax.ShapeDtypeStruct((_NCHUNK, 1), jnp.int32),
            jax.ShapeDtypeStruct((1, 1), jnp.int32),
            jax.ShapeDtypeStruct((1, 1), jnp.int32),
        ],
    )(probs2d, gumbel2d)


_NC = 2
_NSUB = 16
_NW = _NC * _NSUB
_CHUNK = _VPAD // _NW
_CAP = _NS
_B = 20480
_PPW = _B // _NW
_NPW = _NS // _NW


def _compact_body(keys_hbm, meta_hbm, out_hbm, keys_v, vals_v, idx_v, meta_v,
                  sem):
    w = jax.lax.axis_index("s") * _NC + jax.lax.axis_index("c")
    base = w * _CHUNK
    pltpu.sync_copy(keys_hbm.at[pl.ds(base, _CHUNK)], keys_v)
    pltpu.sync_copy(meta_hbm, meta_v)
    lanes = jax.lax.iota(jnp.int32, 16)
    m0 = meta_v[pl.ds(0, 16)]
    m1 = meta_v[pl.ds(16, 16)]
    m2 = meta_v[pl.ds(32, 16)]
    tval = jnp.sum(jnp.where(lanes == 0, m2, 0))
    ival = jnp.sum(jnp.where(lanes == 1, m2, 0))
    wv = jnp.full((16,), w, jnp.int32)
    offsel = jnp.where(wv < 16, m0, m1)
    off = jnp.sum(jnp.where(lanes == (w % 16), offsel, 0))
    tvec = jnp.full((16,), tval, jnp.int32)
    ivec = jnp.full((16,), ival, jnp.int32)

    def body(i, cnt):
        k = keys_v[pl.ds(i * 16, 16)]
        gidx = base + i * 16 + lanes
        sel = (k > tvec) | ((k == tvec) & (gidx <= ivec))
        seli = sel.astype(jnp.int32)
        ranks = plsc.cumsum(seli) - 1 + cnt
        plsc.store_scatter(vals_v, [ranks], gidx, mask=sel)
        return cnt + jnp.sum(seli)

    cnt = jax.lax.fori_loop(0, _CHUNK // 16, body, jnp.int32(0))

    cntv = jnp.full((16,), cnt, jnp.int32)
    offv = jnp.full((16,), off, jnp.int32)
    for j in range(_CAP // 128):
        for c in range(8):
            p = j * 128 + c * 16 + lanes
            idx_v[j, pl.ds(c * 16, 16)] = jnp.where(
                p < cntv, offv + p, _NS + w
            )
    cps = []
    for j in range(_CAP // 128):
        cps.append(pltpu.async_copy(
            vals_v.at[pl.ds(j * 128, 128)], out_hbm.at[idx_v.at[j]], sem
        ))
    for cp in cps:
        cp.wait()


def _compact(keys_flat, meta):
    mesh = plsc.VectorSubcoreMesh(core_axis_name="c", subcore_axis_name="s")
    f = pl.kernel(
        _compact_body,
        out_type=jax.ShapeDtypeStruct((_NS + _NW,), jnp.int32),
        mesh=mesh,
        compiler_params=pltpu.CompilerParams(needs_layout_passes=False),
        scratch_types=[
            pltpu.VMEM((_CHUNK,), jnp.int32),
            pltpu.VMEM((_CAP,), jnp.int32),
            pltpu.VMEM((_CAP // 128, 128), jnp.int32),
            pltpu.VMEM((64,), jnp.int32),
            pltpu.SemaphoreType.DMA,
        ],
    )
    return f(keys_flat, meta)


def _gather_body(emb_hbm, yf_hbm, sid_hbm, epos_hbm, eneg_hbm,
                 yidx_v, sidx_v, prow_v, nrow_v, sem):
    w = jax.lax.axis_index("s") * _NC + jax.lax.axis_index("c")
    pb = w * _PPW
    nb = w * _NPW
    for c in range(_PPW // 128):
        pltpu.sync_copy(yf_hbm.at[pl.ds(pb + c * 128, 128)], yidx_v.at[c])
    pltpu.sync_copy(sid_hbm.at[pl.ds(nb, _NPW)],
                    sidx_v.at[0, pl.ds(0, _NPW)])
    pltpu.sync_copy(sid_hbm.at[pl.ds(nb, _NPW)],
                    sidx_v.at[0, pl.ds(_NPW, _NPW)])
    cps = []
    for c in range(_PPW // 128):
        cps.append(pltpu.async_copy(
            emb_hbm.at[yidx_v.at[c]], prow_v.at[pl.ds(c * 128, 128)], sem))
    cps.append(pltpu.async_copy(emb_hbm.at[sidx_v.at[0]], nrow_v, sem))
    for cp in cps:
        cp.wait()
    pltpu.sync_copy(prow_v, epos_hbm.at[pl.ds(pb, _PPW)])
    pltpu.sync_copy(nrow_v.at[pl.ds(0, _NPW)], eneg_hbm.at[pl.ds(nb, _NPW)])


def _gathers(emb_table, yf, sid):
    mesh = plsc.VectorSubcoreMesh(core_axis_name="c", subcore_axis_name="s")
    f = pl.kernel(
        _gather_body,
        compiler_params=pltpu.CompilerParams(
            needs_layout_passes=False, use_tc_tiling_on_sc=False),
        out_type=[
            jax.ShapeDtypeStruct((_B, _D), jnp.float32),
            jax.ShapeDtypeStruct((_NS, _D), jnp.float32),
        ],
        mesh=mesh,
        scratch_types=[
            pltpu.VMEM((_PPW // 128, 128), jnp.int32),
            pltpu.VMEM((1, 128), jnp.int32),
            pltpu.VMEM((_PPW, _D), jnp.float32),
            pltpu.VMEM((128, _D), jnp.float32),
            pltpu.SemaphoreType.DMA,
        ],
    )
    return f(emb_table, yf, sid)


def _loss_body(h_ref, epos_ref, yf_ref, tp_ref, eneg_ref, sid_ref, sp_ref,
               loss_ref, acc_ref, cnt_ref):
    step = pl.program_id(0)

    @pl.when(step == 0)
    def _():
        acc_ref[0, 0] = 0.0
        cnt_ref[0, 0] = 0.0

    h = h_ref[...]
    eneg = eneg_ref[...]
    neg = jax.lax.dot_general(
        h, eneg, (((1,), (1,)), ((), ())), preferred_element_type=jnp.float32
    )
    yf = yf_ref[...]
    sid = sid_ref[...]
    logsp = jnp.log(sp_ref[...] + 1e-10)
    negl = jnp.where(yf == sid, -1e9, neg) - logsp
    posl = (jnp.sum(h * epos_ref[...], axis=1, keepdims=True)
            - jnp.log(tp_ref[...] + 1e-10))
    m = jnp.maximum(jnp.max(negl, axis=1, keepdims=True), posl)
    s = jnp.sum(jnp.exp(negl - m), axis=1, keepdims=True) + jnp.exp(posl - m)
    per_row = m + jnp.log(s) - posl
    valid = yf != 0
    acc_ref[0, 0] += jnp.sum(jnp.where(valid, per_row, 0.0))
    cnt_ref[0, 0] += jnp.sum(valid.astype(jnp.float32))

    @pl.when(step == pl.num_programs(0) - 1)
    def _():
        loss_ref[...] = jnp.full((1, 1), acc_ref[0, 0] / cnt_ref[0, 0],
                                 dtype=jnp.float32)


def _fused_loss(h, epos, yf, tp, eneg, sid, sp):
    n = h.shape[0]
    grid = n // _BR
    return pl.pallas_call(
        _loss_body,
        grid=(grid,),
        in_specs=[
            pl.BlockSpec((_BR, _D), lambda i: (i, 0)),
            pl.BlockSpec((_BR, _D), lambda i: (i, 0)),
            pl.BlockSpec((_BR, 1), lambda i: (i, 0)),
            pl.BlockSpec((_BR, 1), lambda i: (i, 0)),
            pl.BlockSpec((_NS, _D), lambda i: (0, 0)),
            pl.BlockSpec((1, _NS), lambda i: (0, 0)),
            pl.BlockSpec((1, _NS), lambda i: (0, 0)),
        ],
        out_specs=pl.BlockSpec((1, 1), lambda i: (0, 0)),
        out_shape=jax.ShapeDtypeStruct((1, 1), jnp.float32),
        scratch_shapes=[
            pltpu.SMEM((1, 1), jnp.float32),
            pltpu.SMEM((1, 1), jnp.float32),
        ],
    )(h, epos, yf, tp, eneg, sid, sp)


def kernel(hidden, y, emb_table, sampling_probs):
    h = hidden.reshape(-1, _D)
    yf = y.reshape(-1).astype(jnp.int32)
    probs_pad = jnp.zeros((_VPAD,), jnp.float32).at[:_VOCAB].set(sampling_probs)
    keys, offs, tt, ti = _select(
        probs_pad.reshape(_ROWS, _COLS),
        _gumbel_padded().reshape(_ROWS, _COLS),
    )
    meta = jnp.concatenate([
        offs.reshape(-1), tt.reshape(-1), ti.reshape(-1),
        jnp.zeros((30,), jnp.int32),
    ])
    sid_full = _compact(keys.reshape(-1), meta)
    sid = sid_full[:_NS]
    epos = emb_table[yf]
    eneg = emb_table[sid]
    tp = sampling_probs[yf]
    sp = sampling_probs[sid]
    loss = _fused_loss(
        h, epos, yf.reshape(-1, 1), tp.reshape(-1, 1),
        eneg, sid.reshape(1, -1), sp.reshape(1, -1),
    )
    return loss[0, 0]


# speedup vs baseline: 1.0020x; 1.0020x over previous
import jax
import jax.numpy as jnp
from jax.experimental import pallas as pl
from jax.experimental.pallas import tpu as pltpu
from jax.experimental.pallas import tpu_sc as plsc

_VOCAB = 1000000
_D = 64
_NS = 2048
_BR = 1024

_VPAD = 1024 * 1024


def _gumbel_padded():
    u = jax.random.uniform(
        jax.random.key(42), (_VOCAB,), minval=1e-10, maxval=1.0
    )
    g = -jnp.log(-jnp.log(u))
    return jnp.full((_VPAD,), -1e30, jnp.float32).at[:_VOCAB].set(g)


_ROWS = 1024
_COLS = 1024
_NCHUNK = 32


def _select_body(probs_ref, gumbel_ref, keys_ref, offs_ref, tt_ref, ti_ref):
    z = jnp.log(probs_ref[...] + 1e-10) + gumbel_ref[...]
    b = jax.lax.bitcast_convert_type(z, jnp.uint32)
    key = jnp.where((b >> 31) == 1, ~b, b | jnp.uint32(0x80000000))
    keys_ref[...] = jax.lax.bitcast_convert_type(
        key ^ jnp.uint32(0x80000000), jnp.int32
    )

    row = jax.lax.broadcasted_iota(jnp.int32, (_ROWS, _COLS), 0)
    col = jax.lax.broadcasted_iota(jnp.int32, (_ROWS, _COLS), 1)
    idx = row * _COLS + col

    def bit_body(i, t):
        cand = t | jax.lax.shift_left(
            jnp.uint32(1), (31 - i).astype(jnp.uint32)
        )
        cnt = jnp.sum((key >= cand).astype(jnp.int32))
        return jnp.where(cnt >= _NS, cand, t)

    tval = jax.lax.fori_loop(0, 32, bit_body, jnp.uint32(0))

    need = _NS - jnp.sum((key > tval).astype(jnp.int32))

    def tie_body(j, iv):
        cand = iv | jax.lax.shift_left(jnp.int32(1), 19 - j)
        cnt = jnp.sum(((key == tval) & (idx <= cand)).astype(jnp.int32))
        return jnp.where(cnt <= need, cand, iv)

    ival = jax.lax.fori_loop(0, 20, tie_body, jnp.int32(0))

    sel = (key > tval) | ((key == tval) & (idx <= ival))
    rowsum = jnp.sum(sel.astype(jnp.float32), axis=1, keepdims=True)
    amat = (
        (jax.lax.broadcasted_iota(jnp.int32, (_NCHUNK, _ROWS), 1)
         // (_ROWS // _NCHUNK))
        == jax.lax.broadcasted_iota(jnp.int32, (_NCHUNK, _ROWS), 0)
    ).astype(jnp.float32)
    counts = jnp.dot(amat, rowsum, preferred_element_type=jnp.float32)
    ltri = (
        jax.lax.broadcasted_iota(jnp.int32, (_NCHUNK, _NCHUNK), 0)
        > jax.lax.broadcasted_iota(jnp.int32, (_NCHUNK, _NCHUNK), 1)
    ).astype(jnp.float32)
    offs = jnp.dot(ltri, counts, preferred_element_type=jnp.float32)
    offs_ref[...] = offs.astype(jnp.int32)
    t_signed = jax.lax.bitcast_convert_type(
        tval ^ jnp.uint32(0x80000000), jnp.int32
    )
    tt_ref[...] = jnp.full((1, 1), t_signed, dtype=jnp.int32)
    ti_ref[...] = jnp.full((1, 1), ival, dtype=jnp.int32)


def _select(probs2d, gumbel2d):
    return pl.pallas_call(
        _select_body,
        grid=(1,),
        in_specs=[
            pl.BlockSpec((_ROWS, _COLS), lambda i: (0, 0)),
            pl.BlockSpec((_ROWS, _COLS), lambda i: (0, 0)),
        ],
        out_specs=[
            pl.BlockSpec((_ROWS, _COLS), lambda i: (0, 0)),
            pl.BlockSpec((_NCHUNK, 1), lambda i: (0, 0)),
            pl.BlockSpec((1, 1), lambda i: (0, 0)),
            pl.BlockSpec((1, 1), lambda i: (0, 0)),
        ],
        out_shape=[
            jax.ShapeDtypeStruct((_ROWS, _COLS), jnp.int32),
            jax.ShapeDtypeStruct((_NCHUNK, 1), jnp.int32),
            jax.ShapeDtypeStruct((1, 1), jnp.int32),
            jax.ShapeDtypeStruct((1, 1), jnp.int32),
        ],
    )(probs2d, gumbel2d)


_NC = 2
_NSUB = 16
_NW = _NC * _NSUB
_CHUNK = _VPAD // _NW
_CAP = _NS
_B = 20480
_PPW = _B // _NW
_NPW = _NS // _NW


def _compact_body(keys_hbm, meta_hbm, out_hbm, keys_v, vals_v, idx_v, meta_v,
                  sem):
    w = jax.lax.axis_index("s") * _NC + jax.lax.axis_index("c")
    base = w * _CHUNK
    pltpu.sync_copy(keys_hbm.at[pl.ds(base, _CHUNK)], keys_v)
    pltpu.sync_copy(meta_hbm, meta_v)
    lanes = jax.lax.iota(jnp.int32, 16)
    m0 = meta_v[pl.ds(0, 16)]
    m1 = meta_v[pl.ds(16, 16)]
    m2 = meta_v[pl.ds(32, 16)]
    tval = jnp.sum(jnp.where(lanes == 0, m2, 0))
    ival = jnp.sum(jnp.where(lanes == 1, m2, 0))
    wv = jnp.full((16,), w, jnp.int32)
    offsel = jnp.where(wv < 16, m0, m1)
    off = jnp.sum(jnp.where(lanes == (w % 16), offsel, 0))
    tvec = jnp.full((16,), tval, jnp.int32)
    ivec = jnp.full((16,), ival, jnp.int32)

    @plsc.parallel_loop(0, _CHUNK // 16, unroll=8, carry=jnp.int32(0))
    def _scan(i, cnt):
        k = keys_v[pl.ds(i * 16, 16)]
        gidx = base + i * 16 + lanes
        sel = (k > tvec) | ((k == tvec) & (gidx <= ivec))
        seli = sel.astype(jnp.int32)
        c = plsc.cumsum(seli)
        plsc.store_scatter(vals_v, [c - 1 + cnt], gidx, mask=sel)
        return cnt + jnp.sum(seli)

    cnt = _scan

    cntv = jnp.full((16,), cnt, jnp.int32)
    offv = jnp.full((16,), off, jnp.int32)
    for j in range(_CAP // 128):
        for c in range(8):
            p = j * 128 + c * 16 + lanes
            idx_v[j, pl.ds(c * 16, 16)] = jnp.where(
                p < cntv, offv + p, _NS + w
            )
    cps = []
    for j in range(_CAP // 128):
        cps.append(pltpu.async_copy(
            vals_v.at[pl.ds(j * 128, 128)], out_hbm.at[idx_v.at[j]], sem
        ))
    for cp in cps:
        cp.wait()


def _compact(keys_flat, meta):
    mesh = plsc.VectorSubcoreMesh(core_axis_name="c", subcore_axis_name="s")
    f = pl.kernel(
        _compact_body,
        out_type=jax.ShapeDtypeStruct((_NS + _NW,), jnp.int32),
        mesh=mesh,
        compiler_params=pltpu.CompilerParams(needs_layout_passes=False),
        scratch_types=[
            pltpu.VMEM((_CHUNK,), jnp.int32),
            pltpu.VMEM((_CAP,), jnp.int32),
            pltpu.VMEM((_CAP // 128, 128), jnp.int32),
            pltpu.VMEM((64,), jnp.int32),
            pltpu.SemaphoreType.DMA,
        ],
    )
    return f(keys_flat, meta)


def _gather_body(emb_hbm, yf_hbm, sid_hbm, epos_hbm, eneg_hbm,
                 yidx_v, sidx_v, prow_v, nrow_v, sem):
    w = jax.lax.axis_index("s") * _NC + jax.lax.axis_index("c")
    pb = w * _PPW
    nb = w * _NPW
    for c in range(_PPW // 128):
        pltpu.sync_copy(yf_hbm.at[pl.ds(pb + c * 128, 128)], yidx_v.at[c])
    pltpu.sync_copy(sid_hbm.at[pl.ds(nb, _NPW)],
                    sidx_v.at[0, pl.ds(0, _NPW)])
    pltpu.sync_copy(sid_hbm.at[pl.ds(nb, _NPW)],
                    sidx_v.at[0, pl.ds(_NPW, _NPW)])
    cps = []
    for c in range(_PPW // 128):
        cps.append(pltpu.async_copy(
            emb_hbm.at[yidx_v.at[c]], prow_v.at[pl.ds(c * 128, 128)], sem))
    cps.append(pltpu.async_copy(emb_hbm.at[sidx_v.at[0]], nrow_v, sem))
    for cp in cps:
        cp.wait()
    pltpu.sync_copy(prow_v, epos_hbm.at[pl.ds(pb, _PPW)])
    pltpu.sync_copy(nrow_v.at[pl.ds(0, _NPW)], eneg_hbm.at[pl.ds(nb, _NPW)])


def _gathers(emb_table, yf, sid):
    mesh = plsc.VectorSubcoreMesh(core_axis_name="c", subcore_axis_name="s")
    f = pl.kernel(
        _gather_body,
        compiler_params=pltpu.CompilerParams(
            needs_layout_passes=False, use_tc_tiling_on_sc=False),
        out_type=[
            jax.ShapeDtypeStruct((_B, _D), jnp.float32),
            jax.ShapeDtypeStruct((_NS, _D), jnp.float32),
        ],
        mesh=mesh,
        scratch_types=[
            pltpu.VMEM((_PPW // 128, 128), jnp.int32),
            pltpu.VMEM((1, 128), jnp.int32),
            pltpu.VMEM((_PPW, _D), jnp.float32),
            pltpu.VMEM((128, _D), jnp.float32),
            pltpu.SemaphoreType.DMA,
        ],
    )
    return f(emb_table, yf, sid)


def _loss_body(h_ref, epos_ref, yf_ref, tp_ref, eneg_ref, sid_ref, sp_ref,
               loss_ref, acc_ref, cnt_ref):
    step = pl.program_id(0)

    @pl.when(step == 0)
    def _():
        acc_ref[0, 0] = 0.0
        cnt_ref[0, 0] = 0.0

    h = h_ref[...]
    eneg = eneg_ref[...]
    neg = jax.lax.dot_general(
        h, eneg, (((1,), (1,)), ((), ())), preferred_element_type=jnp.float32
    )
    yf = yf_ref[...]
    sid = sid_ref[...]
    logsp = jnp.log(sp_ref[...] + 1e-10)
    negl = jnp.where(yf == sid, -1e9, neg) - logsp
    posl = (jnp.sum(h * epos_ref[...], axis=1, keepdims=True)
            - jnp.log(tp_ref[...] + 1e-10))
    m = jnp.maximum(jnp.max(negl, axis=1, keepdims=True), posl)
    s = jnp.sum(jnp.exp(negl - m), axis=1, keepdims=True) + jnp.exp(posl - m)
    per_row = m + jnp.log(s) - posl
    valid = yf != 0
    acc_ref[0, 0] += jnp.sum(jnp.where(valid, per_row, 0.0))
    cnt_ref[0, 0] += jnp.sum(valid.astype(jnp.float32))

    @pl.when(step == pl.num_programs(0) - 1)
    def _():
        loss_ref[...] = jnp.full((1, 1), acc_ref[0, 0] / cnt_ref[0, 0],
                                 dtype=jnp.float32)


def _fused_loss(h, epos, yf, tp, eneg, sid, sp):
    n = h.shape[0]
    grid = n // _BR
    return pl.pallas_call(
        _loss_body,
        grid=(grid,),
        in_specs=[
            pl.BlockSpec((_BR, _D), lambda i: (i, 0)),
            pl.BlockSpec((_BR, _D), lambda i: (i, 0)),
            pl.BlockSpec((_BR, 1), lambda i: (i, 0)),
            pl.BlockSpec((_BR, 1), lambda i: (i, 0)),
            pl.BlockSpec((_NS, _D), lambda i: (0, 0)),
            pl.BlockSpec((1, _NS), lambda i: (0, 0)),
            pl.BlockSpec((1, _NS), lambda i: (0, 0)),
        ],
        out_specs=pl.BlockSpec((1, 1), lambda i: (0, 0)),
        out_shape=jax.ShapeDtypeStruct((1, 1), jnp.float32),
        scratch_shapes=[
            pltpu.SMEM((1, 1), jnp.float32),
            pltpu.SMEM((1, 1), jnp.float32),
        ],
    )(h, epos, yf, tp, eneg, sid, sp)


def kernel(hidden, y, emb_table, sampling_probs):
    h = hidden.reshape(-1, _D)
    yf = y.reshape(-1).astype(jnp.int32)
    probs_pad = jnp.zeros((_VPAD,), jnp.float32).at[:_VOCAB].set(sampling_probs)
    keys, offs, tt, ti = _select(
        probs_pad.reshape(_ROWS, _COLS),
        _gumbel_padded().reshape(_ROWS, _COLS),
    )
    meta = jnp.concatenate([
        offs.reshape(-1), tt.reshape(-1), ti.reshape(-1),
        jnp.zeros((30,), jnp.int32),
    ])
    sid_full = _compact(keys.reshape(-1), meta)
    sid = sid_full[:_NS]
    epos = emb_table[yf]
    eneg = emb_table[sid]
    tp = sampling_probs[yf]
    sp = sampling_probs[sid]
    loss = _fused_loss(
        h, epos, yf.reshape(-1, 1), tp.reshape(-1, 1),
        eneg, sid.reshape(1, -1), sp.reshape(1, -1),
    )
    return loss[0, 0]
